# Initial kernel scaffold; baseline (speedup 1.0000x reference)
import jax
import jax.numpy as jnp
from jax.experimental import pallas as pl


def kernel(x, table):
    raise NotImplementedError("write your pallas kernel here")


# speedup vs baseline: 4.7296x; 4.7296x over previous
import functools

import jax
import jax.numpy as jnp
from jax import lax
from jax.experimental import pallas as pl
from jax.experimental.pallas import tpu as pltpu
from jax.experimental.pallas import tpu_sc as plsc

_D = 64
_L = 128
_NK = 4
_NBUF = 2


@functools.lru_cache(maxsize=None)
def _build(num_rows: int):
    info = plsc.get_sparse_core_info()
    nw = info.num_cores * info.num_subcores
    rows_per_w = num_rows // nw
    n_chunks = rows_per_w // _NK
    assert num_rows % nw == 0 and rows_per_w % _NK == 0 and n_chunks % 2 == 0

    mesh = plsc.VectorSubcoreMesh(core_axis_name="c", subcore_axis_name="s")

    @functools.partial(
        pl.kernel,
        mesh=mesh,
        compiler_params=pltpu.CompilerParams(use_tc_tiling_on_sc=False),
        out_type=jax.ShapeDtypeStruct((num_rows, _L, _D), jnp.float32),
        scratch_types=[
            pltpu.VMEM((_NBUF, _NK, _L), jnp.int32),
            pltpu.VMEM((_NBUF, _NK, _L, _D), jnp.float32),
            pltpu.SemaphoreType.DMA,
            pltpu.SemaphoreType.DMA,
            pltpu.SemaphoreType.DMA,
            pltpu.SemaphoreType.DMA,
        ],
    )
    def gather_kernel(x_hbm, table_hbm, out_hbm, idx_v, rows_v,
                      g0, g1, s0, s1):
        cid = lax.axis_index("c")
        sid = lax.axis_index("s")
        wid = sid * info.num_cores + cid
        base = wid * rows_per_w
        gsems = (g0, g1)
        ssems = (s0, s1)

        def fire(b, ci):
            row0 = base + ci * _NK
            pltpu.sync_copy(x_hbm.at[pl.ds(row0, _NK)], idx_v.at[b])
            for j in range(_NK):
                pltpu.async_copy(table_hbm.at[idx_v.at[b].at[j]],
                                 rows_v.at[b].at[j], gsems[b])

        def drain_gathers(b, ci):
            row0 = base + ci * _NK
            pltpu.make_async_copy(out_hbm.at[pl.ds(row0, _NK)],
                                  rows_v.at[b], gsems[b]).wait()

        def store(b, ci):
            row0 = base + ci * _NK
            pltpu.async_copy(rows_v.at[b], out_hbm.at[pl.ds(row0, _NK)],
                             ssems[b])

        def drain_store(b, ci):
            row0 = base + ci * _NK
            pltpu.make_async_copy(rows_v.at[b],
                                  out_hbm.at[pl.ds(row0, _NK)],
                                  ssems[b]).wait()

        for b in range(_NBUF):
            fire(b, b)

        def body(i, carry):
            for b in range(_NBUF):
                ci = i * _NBUF + b
                drain_gathers(b, ci)
                store(b, ci)
                drain_store(b, ci)
                fire(b, ci + _NBUF)
            return carry

        lax.fori_loop(0, (n_chunks - _NBUF) // _NBUF, body, 0)

        for b in range(_NBUF):
            ci = n_chunks - _NBUF + b
            drain_gathers(b, ci)
            store(b, ci)
        for b in range(_NBUF):
            drain_store(b, n_chunks - _NBUF + b)

    return gather_kernel


def kernel(x, table):
    s0, s1 = x.shape
    total = s0 * s1
    num_rows = total // _L
    x_rows = x.reshape(num_rows, _L).astype(jnp.int32)
    out = _build(num_rows)(x_rows, table)
    return out.reshape(s0, s1, _D)


# speedup vs baseline: 4.7907x; 1.0129x over previous
import functools

import jax
import jax.numpy as jnp
from jax import lax
from jax.experimental import pallas as pl
from jax.experimental.pallas import tpu as pltpu
from jax.experimental.pallas import tpu_sc as plsc

_D = 64
_L = 128
_NK = 5
_NBUF = 2


@functools.lru_cache(maxsize=None)
def _build(num_rows: int):
    info = plsc.get_sparse_core_info()
    nw = info.num_cores * info.num_subcores
    rows_per_w = num_rows // nw
    n_chunks = rows_per_w // _NK
    assert num_rows % nw == 0 and rows_per_w % _NK == 0 and n_chunks % 2 == 0

    mesh = plsc.VectorSubcoreMesh(core_axis_name="c", subcore_axis_name="s")

    @functools.partial(
        pl.kernel,
        mesh=mesh,
        compiler_params=pltpu.CompilerParams(use_tc_tiling_on_sc=False),
        out_type=jax.ShapeDtypeStruct((num_rows, _L, _D), jnp.float32),
        scratch_types=[
            pltpu.VMEM((_NBUF, _NK, _L), jnp.int32),
            pltpu.VMEM((_NBUF, _NK, _L, _D), jnp.float32),
            pltpu.SemaphoreType.DMA,
            pltpu.SemaphoreType.DMA,
            pltpu.SemaphoreType.DMA,
            pltpu.SemaphoreType.DMA,
        ],
    )
    def gather_kernel(x_hbm, table_hbm, out_hbm, idx_v, rows_v,
                      g0, g1, s0, s1):
        cid = lax.axis_index("c")
        sid = lax.axis_index("s")
        wid = sid * info.num_cores + cid
        base = wid * rows_per_w
        gsems = (g0, g1)
        ssems = (s0, s1)

        def fire(b, ci):
            row0 = base + ci * _NK
            pltpu.sync_copy(x_hbm.at[pl.ds(row0, _NK)], idx_v.at[b])
            for j in range(_NK):
                pltpu.async_copy(table_hbm.at[idx_v.at[b].at[j]],
                                 rows_v.at[b].at[j], gsems[b])

        def drain_gathers(b, ci):
            row0 = base + ci * _NK
            pltpu.make_async_copy(out_hbm.at[pl.ds(row0, _NK)],
                                  rows_v.at[b], gsems[b]).wait()

        def store(b, ci):
            row0 = base + ci * _NK
            pltpu.async_copy(rows_v.at[b], out_hbm.at[pl.ds(row0, _NK)],
                             ssems[b])

        def drain_store(b, ci):
            row0 = base + ci * _NK
            pltpu.make_async_copy(rows_v.at[b],
                                  out_hbm.at[pl.ds(row0, _NK)],
                                  ssems[b]).wait()

        for b in range(_NBUF):
            fire(b, b)

        def body(i, carry):
            for b in range(_NBUF):
                ci = i * _NBUF + b
                drain_gathers(b, ci)
                store(b, ci)
                drain_store(b, ci)
                fire(b, ci + _NBUF)
            return carry

        lax.fori_loop(0, (n_chunks - _NBUF) // _NBUF, body, 0)

        for b in range(_NBUF):
            ci = n_chunks - _NBUF + b
            drain_gathers(b, ci)
            store(b, ci)
        for b in range(_NBUF):
            drain_store(b, n_chunks - _NBUF + b)

    return gather_kernel


def kernel(x, table):
    s0, s1 = x.shape
    total = s0 * s1
    num_rows = total // _L
    x_rows = x.reshape(num_rows, _L).astype(jnp.int32)
    out = _build(num_rows)(x_rows, table)
    return out.reshape(s0, s1, _D)
